# baseline (speedup 1.0000x reference)
import jax
import jax.numpy as jnp
from jax.experimental import pallas as pl
from jax.experimental.pallas import tpu as pltpu

_T = 50
_MS = 4096
_MSEM = 4096
_CTX = 8192
_SEN = 8192
_K_8192 = int(8192 * 0.05)
_K_4096 = int(4096 * 0.05)

_R = 256
_P1 = _MS // _R
_P2 = _CTX // _R
_P3 = _MSEM // _R
_P4 = _CTX // _R
_S2 = _P1
_S3 = _S2 + _P2
_S4 = _S3 + _P3
_NSTEP = _S4 + _P4


def _masked_mv(w, m):
    return jax.lax.dot_general(
        w.astype(jnp.bfloat16), m.astype(jnp.bfloat16),
        (((1,), (0,)), ((), ())),
        preferred_element_type=jnp.float32)


def _f32_sort_key(x):
    b = jax.lax.bitcast_convert_type(x, jnp.int32)
    return jnp.where(b < 0, b ^ jnp.int32(0x7FFFFFFF), b)


def _topk_mask(scores, k):
    n = scores.shape[0]
    keys = _f32_sort_key(scores.reshape(8, n // 8))
    kk = jnp.int32(k)

    def _cnt_ge(c):
        return jnp.sum((keys >= c).astype(jnp.int32))

    t = jnp.int32(-2147483648)
    for bit in range(30, -2, -2):
        lo = t + (jnp.int32(1) << jnp.int32(bit))
        mid = t + (jnp.int32(1) << jnp.int32(bit + 1))
        hi = mid + (jnp.int32(1) << jnp.int32(bit))
        n_lo, n_mid, n_hi = _cnt_ge(lo), _cnt_ge(mid), _cnt_ge(hi)
        t = jnp.where(n_hi >= kk, hi,
                      jnp.where(n_mid >= kk, mid,
                                jnp.where(n_lo >= kk, lo, t)))

    cnt_gt = jnp.sum((keys > t).astype(jnp.int32))
    need = kk - cnt_gt

    idx = (jax.lax.broadcasted_iota(jnp.int32, (8, n // 8), 0) * (n // 8)
           + jax.lax.broadcasted_iota(jnp.int32, (8, n // 8), 1))
    tie = keys == t
    nbits = max(1, (n - 1).bit_length())
    if nbits % 2:
        nbits += 1

    def _cnt_lt(c):
        return jnp.sum((tie & (idx < c)).astype(jnp.int32))

    m = jnp.int32(0)
    for bit in range(nbits - 2, -2, -2):
        lo = m + (jnp.int32(1) << jnp.int32(bit))
        mid = m + (jnp.int32(1) << jnp.int32(bit + 1))
        hi = mid + (jnp.int32(1) << jnp.int32(bit))
        q_lo, q_mid, q_hi = _cnt_lt(lo), _cnt_lt(mid), _cnt_lt(hi)
        m = jnp.where(q_hi < need, hi,
                      jnp.where(q_mid < need, mid,
                                jnp.where(q_lo < need, lo, m)))

    mask = (keys > t) | (tie & (idx <= m))
    return mask.astype(jnp.float32).reshape(n)


def _chain_body(x_ref, w1_ref, w2_ref, w3_ref, cb_ref, cim_ref, bb_ref,
                bim_ref,
                ctx_hat_ref, ctx_ref, mtl_ref,
                mask1, mask2, mask3, mask4, v1s, us, v2s, chs):
    i = pl.program_id(0)

    @pl.when(i == 0)
    def _():
        mask1[...] = _topk_mask(x_ref[0, 0], _K_8192)

    @pl.when(i < _S2)
    def _():
        v1s[pl.ds(i * _R, _R)] = _masked_mv(w1_ref[...], mask1[...])

    @pl.when(i == _S2)
    def _():
        mk = _topk_mask(v1s[...], _K_4096)
        mask2[...] = mk
        mtl_ref[...] = mk

    @pl.when((i >= _S2) & (i < _S3))
    def _():
        j = i - _S2
        us[pl.ds(j * _R, _R)] = _masked_mv(w2_ref[...], mask2[...])

    @pl.when(i == _S3)
    def _():
        mask3[...] = _topk_mask(us[...] + cb_ref[...] * cim_ref[...], _K_8192)

    @pl.when((i >= _S3) & (i < _S4))
    def _():
        j = i - _S3
        v2s[pl.ds(j * _R, _R)] = (_masked_mv(w3_ref[...], mask3[...])
                                  + bb_ref[...] * bim_ref[...])

    @pl.when(i == _S4)
    def _():
        mk = _topk_mask(v2s[...], _K_4096)
        mask4[...] = mk
        mtl_ref[...] = mk

    @pl.when(i >= _S4)
    def _():
        j = i - _S4
        res = us[pl.ds(j * _R, _R)] + _masked_mv(w2_ref[...], mask4[...])
        ctx_hat_ref[...] = res
        chs[pl.ds(j * _R, _R)] = res

    @pl.when(i == _NSTEP - 1)
    def _():
        ctx_ref[...] = _topk_mask(chs[...], _K_8192)


def _clip(v, lo_, hi_):
    return jnp.minimum(jnp.maximum(v, lo_), hi_)


def kernel(input, mtl_sensory_sen, ctx_mtl, mtl_semantic_ctx, ctx_b, ctx_IM,
           mtl_semantic_b, mtl_semantic_IM, sem_noise):
    def x_map(i):
        return (_T - 1, 0, 0)

    def mtl_map(i):
        return (jnp.where(i < _S4, 0, 1),)

    def w1_map(i):
        return (_clip(i, 0, _P1 - 1), 0)

    def w2_map(i):
        left = i < _S3
        blk = jnp.where(left, _clip(i - _S2, 0, _P2 - 1),
                        _clip(i - _S4, 0, _P4 - 1))
        return (blk, jnp.where(left, 0, 1))

    def w3_map(i):
        return (_clip(i - _S3, 0, _P3 - 1), 0)

    def b_map(i):
        return (_clip(i - _S3, 0, _P3 - 1),)

    def p4_map(i):
        return (_clip(i - _S4, 0, _P4 - 1),)

    full = lambda i: (0,)

    ctx_hat, ctx, mtl = pl.pallas_call(
        _chain_body,
        grid=(_NSTEP,),
        in_specs=[
            pl.BlockSpec((1, 1, _SEN), x_map),
            pl.BlockSpec((_R, _SEN), w1_map),
            pl.BlockSpec((_R, _MS), w2_map),
            pl.BlockSpec((_R, _CTX), w3_map),
            pl.BlockSpec((_CTX,), full),
            pl.BlockSpec((_CTX,), full),
            pl.BlockSpec((_R,), b_map),
            pl.BlockSpec((_R,), b_map),
        ],
        out_specs=[
            pl.BlockSpec((_R,), p4_map),
            pl.BlockSpec((_CTX,), full),
            pl.BlockSpec((_MS,), mtl_map),
        ],
        out_shape=[
            jax.ShapeDtypeStruct((_CTX,), jnp.float32),
            jax.ShapeDtypeStruct((_CTX,), jnp.float32),
            jax.ShapeDtypeStruct((_MS + _MSEM,), jnp.float32),
        ],
        scratch_shapes=[
            pltpu.VMEM((_SEN,), jnp.float32),
            pltpu.VMEM((_MS,), jnp.float32),
            pltpu.VMEM((_CTX,), jnp.float32),
            pltpu.VMEM((_MSEM,), jnp.float32),
            pltpu.VMEM((_MS,), jnp.float32),
            pltpu.VMEM((_CTX,), jnp.float32),
            pltpu.VMEM((_MSEM,), jnp.float32),
            pltpu.VMEM((_CTX,), jnp.float32),
        ],
    )(input.reshape(_T, 1, _SEN), mtl_sensory_sen, ctx_mtl, mtl_semantic_ctx,
      ctx_b, ctx_IM, mtl_semantic_b, mtl_semantic_IM)

    return (ctx_hat, ctx, mtl)


# speedup vs baseline: 1.0600x; 1.0600x over previous
import jax
import jax.numpy as jnp
from jax.experimental import pallas as pl
from jax.experimental.pallas import tpu as pltpu

_T = 50
_MS = 4096
_MSEM = 4096
_CTX = 8192
_SEN = 8192
_K_8192 = int(8192 * 0.05)
_K_4096 = int(4096 * 0.05)

_R = 256
_P1 = _MS // _R
_P2 = _CTX // _R
_P3 = _MSEM // _R
_P4 = _CTX // _R
_S2 = _P1
_S3 = _S2 + _P2
_S4 = _S3 + _P3
_NSTEP = _S4 + _P4


def _masked_mv(w, m):
    return jax.lax.dot_general(
        w, m, (((1,), (0,)), ((), ())),
        precision=jax.lax.Precision.DEFAULT,
        preferred_element_type=jnp.float32)


def _f32_sort_key(x):
    b = jax.lax.bitcast_convert_type(x, jnp.int32)
    return jnp.where(b < 0, b ^ jnp.int32(0x7FFFFFFF), b)


def _topk_mask(scores, k):
    n = scores.shape[0]
    keys = _f32_sort_key(scores.reshape(8, n // 8))
    kk = jnp.int32(k)

    def _cnt_ge(c):
        return jnp.sum((keys >= c).astype(jnp.int32))

    t = jnp.int32(-2147483648)
    for bit in range(30, -2, -2):
        lo = t + (jnp.int32(1) << jnp.int32(bit))
        mid = t + (jnp.int32(1) << jnp.int32(bit + 1))
        hi = mid + (jnp.int32(1) << jnp.int32(bit))
        n_lo, n_mid, n_hi = _cnt_ge(lo), _cnt_ge(mid), _cnt_ge(hi)
        t = jnp.where(n_hi >= kk, hi,
                      jnp.where(n_mid >= kk, mid,
                                jnp.where(n_lo >= kk, lo, t)))

    cnt_gt = jnp.sum((keys > t).astype(jnp.int32))
    need = kk - cnt_gt

    idx = (jax.lax.broadcasted_iota(jnp.int32, (8, n // 8), 0) * (n // 8)
           + jax.lax.broadcasted_iota(jnp.int32, (8, n // 8), 1))
    tie = keys == t
    nbits = max(1, (n - 1).bit_length())
    if nbits % 2:
        nbits += 1

    def _cnt_lt(c):
        return jnp.sum((tie & (idx < c)).astype(jnp.int32))

    m = jnp.int32(0)
    for bit in range(nbits - 2, -2, -2):
        lo = m + (jnp.int32(1) << jnp.int32(bit))
        mid = m + (jnp.int32(1) << jnp.int32(bit + 1))
        hi = mid + (jnp.int32(1) << jnp.int32(bit))
        q_lo, q_mid, q_hi = _cnt_lt(lo), _cnt_lt(mid), _cnt_lt(hi)
        m = jnp.where(q_hi < need, hi,
                      jnp.where(q_mid < need, mid,
                                jnp.where(q_lo < need, lo, m)))

    mask = (keys > t) | (tie & (idx <= m))
    return mask.astype(jnp.float32).reshape(n)


def _chain_body(x_ref, w1_ref, w2_ref, w3_ref, cb_ref, cim_ref, bb_ref,
                bim_ref,
                ctx_hat_ref, ctx_ref, ms_mask_ref, msem_mask_ref,
                mask1, mask2, mask3, mask4, v1s, us, v2s, chs):
    i = pl.program_id(0)

    @pl.when(i == 0)
    def _():
        mask1[...] = _topk_mask(x_ref[...], _K_8192)

    @pl.when(i < _S2)
    def _():
        v1s[pl.ds(i * _R, _R)] = _masked_mv(w1_ref[...], mask1[...])

    @pl.when(i == _S2)
    def _():
        mk = _topk_mask(v1s[...], _K_4096)
        mask2[...] = mk
        ms_mask_ref[...] = mk

    @pl.when((i >= _S2) & (i < _S3))
    def _():
        j = i - _S2
        us[pl.ds(j * _R, _R)] = _masked_mv(w2_ref[...], mask2[...])

    @pl.when(i == _S3)
    def _():
        mask3[...] = _topk_mask(us[...] + cb_ref[...] * cim_ref[...], _K_8192)

    @pl.when((i >= _S3) & (i < _S4))
    def _():
        j = i - _S3
        v2s[pl.ds(j * _R, _R)] = (_masked_mv(w3_ref[...], mask3[...])
                                  + bb_ref[...] * bim_ref[...])

    @pl.when(i == _S4)
    def _():
        mk = _topk_mask(v2s[...], _K_4096)
        mask4[...] = mk
        msem_mask_ref[...] = mk

    @pl.when(i >= _S4)
    def _():
        j = i - _S4
        res = us[pl.ds(j * _R, _R)] + _masked_mv(w2_ref[...], mask4[...])
        ctx_hat_ref[...] = res
        chs[pl.ds(j * _R, _R)] = res

    @pl.when(i == _NSTEP - 1)
    def _():
        ctx_ref[...] = _topk_mask(chs[...], _K_8192)


def _clip(v, lo_, hi_):
    return jnp.minimum(jnp.maximum(v, lo_), hi_)


def kernel(input, mtl_sensory_sen, ctx_mtl, mtl_semantic_ctx, ctx_b, ctx_IM,
           mtl_semantic_b, mtl_semantic_IM, sem_noise):
    x_last = input[_T - 1]

    def w1_map(i):
        return (_clip(i, 0, _P1 - 1), 0)

    def w2_map(i):
        left = i < _S3
        blk = jnp.where(left, _clip(i - _S2, 0, _P2 - 1),
                        _clip(i - _S4, 0, _P4 - 1))
        return (blk, jnp.where(left, 0, 1))

    def w3_map(i):
        return (_clip(i - _S3, 0, _P3 - 1), 0)

    def b_map(i):
        return (_clip(i - _S3, 0, _P3 - 1),)

    def p4_map(i):
        return (_clip(i - _S4, 0, _P4 - 1),)

    full = lambda i: (0,)

    ctx_hat, ctx, ms_mask, msem_mask = pl.pallas_call(
        _chain_body,
        grid=(_NSTEP,),
        in_specs=[
            pl.BlockSpec((_SEN,), full),
            pl.BlockSpec((_R, _SEN), w1_map),
            pl.BlockSpec((_R, _MS), w2_map),
            pl.BlockSpec((_R, _CTX), w3_map),
            pl.BlockSpec((_CTX,), full),
            pl.BlockSpec((_CTX,), full),
            pl.BlockSpec((_R,), b_map),
            pl.BlockSpec((_R,), b_map),
        ],
        out_specs=[
            pl.BlockSpec((_R,), p4_map),
            pl.BlockSpec((_CTX,), full),
            pl.BlockSpec((_MS,), full),
            pl.BlockSpec((_MSEM,), full),
        ],
        out_shape=[
            jax.ShapeDtypeStruct((_CTX,), jnp.float32),
            jax.ShapeDtypeStruct((_CTX,), jnp.float32),
            jax.ShapeDtypeStruct((_MS,), jnp.float32),
            jax.ShapeDtypeStruct((_MSEM,), jnp.float32),
        ],
        scratch_shapes=[
            pltpu.VMEM((_SEN,), jnp.float32),
            pltpu.VMEM((_MS,), jnp.float32),
            pltpu.VMEM((_CTX,), jnp.float32),
            pltpu.VMEM((_MSEM,), jnp.float32),
            pltpu.VMEM((_MS,), jnp.float32),
            pltpu.VMEM((_CTX,), jnp.float32),
            pltpu.VMEM((_MSEM,), jnp.float32),
            pltpu.VMEM((_CTX,), jnp.float32),
        ],
    )(x_last, mtl_sensory_sen, ctx_mtl, mtl_semantic_ctx, ctx_b, ctx_IM,
      mtl_semantic_b, mtl_semantic_IM)

    mtl = jnp.concatenate([ms_mask, msem_mask])
    return (ctx_hat, ctx, mtl)
